# baseline (speedup 1.0000x reference)
import functools

import jax
import jax.numpy as jnp
from jax import lax
from jax.experimental import pallas as pl
from jax.experimental.pallas import tpu as pltpu
from jax.experimental.pallas import tpu_sc as plsc

B = 16384
L = 50
EMB = 64
OUT = 1024
VOCAB2 = 500000
LP = 64

NC = 2
NS = 16
NW = NC * NS
B_PER_W = B // NW
CHUNK = 8
N_CHUNKS = B_PER_W // CHUNK
VECS = EMB // 16

_mesh = plsc.VectorSubcoreMesh(core_axis_name="c", subcore_axis_name="s")


@functools.partial(
    pl.kernel,
    mesh=_mesh,
    out_type=jax.ShapeDtypeStruct((B, EMB), jnp.float32),
    scratch_types=[
        pltpu.VMEM((CHUNK, L), jnp.int32),
        pltpu.VMEM((CHUNK, L), jnp.int32),
        pltpu.VMEM((CHUNK, LP), jnp.int32),
        pltpu.VMEM((CHUNK, LP), jnp.int32),
        pltpu.VMEM((CHUNK, L, 2 * EMB), jnp.float32),
        pltpu.VMEM((CHUNK, L, 2 * EMB), jnp.float32),
        pltpu.VMEM((CHUNK, EMB), jnp.float32),
        pltpu.SemaphoreType.DMA,
        pltpu.SemaphoreType.DMA,
    ],
)
def _pool(idxh_hbm, poff_hbm, tbl_hbm, out_hbm,
          idxh_a, idxh_b, poff_a, poff_b, rows_a, rows_b, out_v,
          sem_a, sem_b):
    wid = lax.axis_index("s") * NC + lax.axis_index("c")
    base = wid * B_PER_W

    def stage_and_fire(c, idxh_v, poff_v, rows_v, sem):
        row0 = base + c * CHUNK
        pltpu.sync_copy(idxh_hbm.at[pl.ds(row0, CHUNK), :], idxh_v)
        pltpu.sync_copy(poff_hbm.at[pl.ds(row0, CHUNK), :], poff_v)
        for j in range(CHUNK):
            pltpu.async_copy(tbl_hbm.at[idxh_v.at[j]], rows_v.at[j], sem)

    def wait_all(idxh_v, rows_v, sem):
        for j in range(CHUNK):
            pltpu.make_async_copy(
                tbl_hbm.at[idxh_v.at[j]], rows_v.at[j], sem).wait()

    def accum_and_store(c, poff_v, rows_v):
        row0 = base + c * CHUNK
        for j in range(CHUNK):
            def group(g, accs):
                pvec = poff_v[j, pl.ds(g * 16, 16)]
                for i in range(16):
                    off = pvec[i]
                    accs = tuple(
                        accs[k]
                        + rows_v[j, g * 16 + i,
                                 pl.ds(pl.multiple_of(off + k * 16, 16), 16)]
                        for k in range(VECS)
                    )
                return accs
            acc = lax.fori_loop(
                0, L // 16, group,
                tuple(jnp.zeros((16,), jnp.float32) for _ in range(VECS)),
            )
            ptail = poff_v[j, pl.ds((L // 16) * 16, 16)]
            for i in range(L % 16):
                off = ptail[i]
                acc = tuple(
                    acc[k]
                    + rows_v[j, (L // 16) * 16 + i,
                             pl.ds(pl.multiple_of(off + k * 16, 16), 16)]
                    for k in range(VECS)
                )
            for k in range(VECS):
                out_v[j, pl.ds(k * 16, 16)] = acc[k] * (1.0 / L)
        pltpu.sync_copy(out_v, out_hbm.at[pl.ds(row0, CHUNK), :])

    stage_and_fire(0, idxh_a, poff_a, rows_a, sem_a)

    def pair_body(i, carry):
        stage_and_fire(2 * i + 1, idxh_b, poff_b, rows_b, sem_b)
        wait_all(idxh_a, rows_a, sem_a)
        accum_and_store(2 * i, poff_a, rows_a)

        @pl.when(i < N_CHUNKS // 2 - 1)
        def _():
            stage_and_fire(2 * i + 2, idxh_a, poff_a, rows_a, sem_a)

        wait_all(idxh_b, rows_b, sem_b)
        accum_and_store(2 * i + 1, poff_b, rows_b)
        return carry

    lax.fori_loop(0, N_CHUNKS // 2, pair_body, 0)


BM = 1024


def _mm_body(m_ref, w_ref, b_ref, o_ref):
    o_ref[...] = jnp.tanh(
        jnp.dot(m_ref[...], w_ref[...], preferred_element_type=jnp.float32)
        + b_ref[...]
    )


def _matmul(m, w, b2d):
    return pl.pallas_call(
        _mm_body,
        grid=(B // BM,),
        in_specs=[
            pl.BlockSpec((BM, EMB), lambda i: (i, 0)),
            pl.BlockSpec((EMB, OUT), lambda i: (0, 0)),
            pl.BlockSpec((1, OUT), lambda i: (0, 0)),
        ],
        out_specs=pl.BlockSpec((BM, OUT), lambda i: (i, 0)),
        out_shape=jax.ShapeDtypeStruct((B, OUT), jnp.float32),
    )(m, w, b2d)


def kernel(input_a, emb_table, W, b):
    tbl2 = emb_table.reshape(VOCAB2, 2 * EMB)
    idx_half = jnp.right_shift(input_a, 1)
    poff = jnp.left_shift(jnp.bitwise_and(input_a, 1), 6)
    poff = jnp.pad(poff, ((0, 0), (0, LP - L)))
    m = _pool(idx_half, poff, tbl2)
    return _matmul(m, W, b.reshape(1, OUT))


# speedup vs baseline: 1.4865x; 1.4865x over previous
import functools

import jax
import jax.numpy as jnp
from jax import lax
from jax.experimental import pallas as pl
from jax.experimental.pallas import tpu as pltpu
from jax.experimental.pallas import tpu_sc as plsc

B = 16384
L = 50
EMB = 64
OUT = 1024
HALF = 512000
LP = 64

NC = 2
NS = 16
NW = NC * NS
B_PER_W = B // NW
CHUNK = 8
N_CHUNKS = B_PER_W // CHUNK
VECS = EMB // 16

_mesh = plsc.VectorSubcoreMesh(core_axis_name="c", subcore_axis_name="s")


@functools.partial(
    pl.kernel,
    mesh=_mesh,
    out_type=jax.ShapeDtypeStruct((B, EMB), jnp.float32),
    scratch_types=[
        pltpu.VMEM((CHUNK, L), jnp.int32),
        pltpu.VMEM((CHUNK, L), jnp.int32),
        pltpu.VMEM((CHUNK, LP), jnp.int32),
        pltpu.VMEM((CHUNK, LP), jnp.int32),
        pltpu.VMEM((CHUNK, L, 2 * EMB), jnp.float32),
        pltpu.VMEM((CHUNK, L, 2 * EMB), jnp.float32),
        pltpu.VMEM((CHUNK, EMB), jnp.float32),
        pltpu.SemaphoreType.DMA,
        pltpu.SemaphoreType.DMA,
    ],
)
def _pool(idxh_hbm, poff_hbm, tbl_hbm, out_hbm,
          idxh_a, idxh_b, poff_a, poff_b, rows_a, rows_b, out_v,
          sem_a, sem_b):
    wid = lax.axis_index("s") * NC + lax.axis_index("c")
    base = wid * B_PER_W

    def stage_and_fire(c, idxh_v, poff_v, rows_v, sem):
        row0 = base + c * CHUNK
        pltpu.sync_copy(idxh_hbm.at[pl.ds(row0, CHUNK), :], idxh_v)
        pltpu.sync_copy(poff_hbm.at[pl.ds(row0, CHUNK), :], poff_v)
        for j in range(CHUNK):
            pltpu.async_copy(tbl_hbm.at[idxh_v.at[j]], rows_v.at[j], sem)

    def wait_all(idxh_v, rows_v, sem):
        for j in range(CHUNK):
            pltpu.make_async_copy(
                tbl_hbm.at[idxh_v.at[j]], rows_v.at[j], sem).wait()

    def accum_and_store(c, poff_v, rows_v):
        row0 = base + c * CHUNK
        for j in range(CHUNK):
            def group(g, accs):
                pvec = poff_v[j, pl.ds(g * 16, 16)]
                for i in range(16):
                    off = pvec[i]
                    accs = tuple(
                        accs[k]
                        + rows_v[j, g * 16 + i,
                                 pl.ds(pl.multiple_of(off + k * 16, 16), 16)]
                        for k in range(VECS)
                    )
                return accs
            acc = lax.fori_loop(
                0, L // 16, group,
                tuple(jnp.zeros((16,), jnp.float32) for _ in range(VECS)),
            )
            ptail = poff_v[j, pl.ds((L // 16) * 16, 16)]
            for i in range(L % 16):
                off = ptail[i]
                acc = tuple(
                    acc[k]
                    + rows_v[j, (L // 16) * 16 + i,
                             pl.ds(pl.multiple_of(off + k * 16, 16), 16)]
                    for k in range(VECS)
                )
            for k in range(VECS):
                out_v[j, pl.ds(k * 16, 16)] = acc[k] * (1.0 / L)
        pltpu.sync_copy(out_v, out_hbm.at[pl.ds(row0, CHUNK), :])

    stage_and_fire(0, idxh_a, poff_a, rows_a, sem_a)

    def pair_body(i, carry):
        stage_and_fire(2 * i + 1, idxh_b, poff_b, rows_b, sem_b)
        wait_all(idxh_a, rows_a, sem_a)
        accum_and_store(2 * i, poff_a, rows_a)

        @pl.when(i < N_CHUNKS // 2 - 1)
        def _():
            stage_and_fire(2 * i + 2, idxh_a, poff_a, rows_a, sem_a)

        wait_all(idxh_b, rows_b, sem_b)
        accum_and_store(2 * i + 1, poff_b, rows_b)
        return carry

    lax.fori_loop(0, N_CHUNKS // 2, pair_body, 0)


BN = 4096


def _tr_body(t1_ref, t2_ref, o_ref):
    o_ref[...] = jnp.concatenate([t1_ref[...].T, t2_ref[...].T], axis=1)


def _repack(tbl_t):
    return pl.pallas_call(
        _tr_body,
        grid=(HALF // BN,),
        in_specs=[
            pl.BlockSpec((EMB, BN), lambda i: (0, i)),
            pl.BlockSpec(
                (EMB, BN),
                lambda i: (0, jnp.minimum(i + HALF // BN, 1000000 // BN)),
            ),
        ],
        out_specs=pl.BlockSpec((BN, 2 * EMB), lambda i: (i, 0)),
        out_shape=jax.ShapeDtypeStruct((HALF, 2 * EMB), jnp.float32),
    )(tbl_t, tbl_t)


BM = 1024


def _mm_body(m_ref, w_ref, b_ref, o_ref):
    o_ref[...] = jnp.tanh(
        jnp.dot(m_ref[...], w_ref[...], preferred_element_type=jnp.float32)
        + b_ref[...]
    )


def _matmul(m, w, b2d):
    return pl.pallas_call(
        _mm_body,
        grid=(B // BM,),
        in_specs=[
            pl.BlockSpec((BM, EMB), lambda i: (i, 0)),
            pl.BlockSpec((EMB, OUT), lambda i: (0, 0)),
            pl.BlockSpec((1, OUT), lambda i: (0, 0)),
        ],
        out_specs=pl.BlockSpec((BM, OUT), lambda i: (i, 0)),
        out_shape=jax.ShapeDtypeStruct((B, OUT), jnp.float32),
    )(m, w, b2d)


def kernel(input_a, emb_table, W, b):
    tbl2 = _repack(emb_table.T)
    hi = (input_a >= HALF).astype(jnp.int32)
    idx_half = input_a - hi * HALF
    poff = jnp.pad(hi << 6, ((0, 0), (0, LP - L)))
    m = _pool(idx_half, poff, tbl2)
    return _matmul(m, W, b.reshape(1, OUT))


# speedup vs baseline: 1.7054x; 1.1473x over previous
import functools

import jax
import jax.numpy as jnp
from jax import lax
from jax.experimental import pallas as pl
from jax.experimental.pallas import tpu as pltpu
from jax.experimental.pallas import tpu_sc as plsc

B = 16384
L = 50
EMB = 64
OUT = 1024
HALF = 512000
LP = 64

NC = 2
NS = 16
NW = NC * NS
B_PER_W = B // NW
CHUNK = 4
IBLK = 128
N_BLK = B_PER_W // IBLK
CPB = IBLK // CHUNK
VECS = EMB // 16

_mesh = plsc.VectorSubcoreMesh(core_axis_name="c", subcore_axis_name="s")


@functools.partial(
    pl.kernel,
    mesh=_mesh,
    out_type=jax.ShapeDtypeStruct((B, EMB), jnp.float32),
    scratch_types=[
        pltpu.VMEM((IBLK, L), jnp.int32),
        pltpu.VMEM((IBLK, LP), jnp.int32),
        pltpu.VMEM((CHUNK, L, 2 * EMB), jnp.float32),
        pltpu.VMEM((CHUNK, L, 2 * EMB), jnp.float32),
        pltpu.VMEM((CHUNK, EMB), jnp.float32),
        pltpu.VMEM((CHUNK, EMB), jnp.float32),
        pltpu.SemaphoreType.DMA,
        pltpu.SemaphoreType.DMA,
        pltpu.SemaphoreType.DMA,
        pltpu.SemaphoreType.DMA,
    ],
)
def _pool(idxh_hbm, poff_hbm, tbl_hbm, out_hbm,
          idxh_blk, poff_blk, rows_a, rows_b, out_a, out_b,
          sem_a, sem_b, sem_oa, sem_ob):
    wid = lax.axis_index("s") * NC + lax.axis_index("c")
    base = wid * B_PER_W

    def fire(c, rows_v, sem):
        for j in range(CHUNK):
            pltpu.async_copy(
                tbl_hbm.at[idxh_blk.at[c * CHUNK + j]], rows_v.at[j], sem)

    def wait_gather(c, rows_v, sem):
        for j in range(CHUNK):
            pltpu.make_async_copy(
                tbl_hbm.at[idxh_blk.at[c * CHUNK + j]], rows_v.at[j],
                sem).wait()

    def out_copy(row0, c, out_v, sem):
        return pltpu.make_async_copy(
            out_v, out_hbm.at[pl.ds(row0 + c * CHUNK, CHUNK), :], sem)

    def accum(c, rows_v, out_v):
        for j in range(CHUNK):
            r = c * CHUNK + j

            def group(g, accs):
                pvec = poff_blk[r, pl.ds(g * 16, 16)]
                for i in range(16):
                    off = pvec[i]
                    accs = tuple(
                        accs[k]
                        + rows_v[j, g * 16 + i,
                                 pl.ds(pl.multiple_of(off + k * 16, 16), 16)]
                        for k in range(VECS)
                    )
                return accs
            acc = lax.fori_loop(
                0, L // 16, group,
                tuple(jnp.zeros((16,), jnp.float32) for _ in range(VECS)),
            )
            ptail = poff_blk[r, pl.ds((L // 16) * 16, 16)]
            for i in range(L % 16):
                off = ptail[i]
                acc = tuple(
                    acc[k]
                    + rows_v[j, (L // 16) * 16 + i,
                             pl.ds(pl.multiple_of(off + k * 16, 16), 16)]
                    for k in range(VECS)
                )
            for k in range(VECS):
                out_v[j, pl.ds(k * 16, 16)] = acc[k] * (1.0 / L)

    def block_body(blk, carry):
        row0 = base + blk * IBLK
        pltpu.sync_copy(idxh_hbm.at[pl.ds(row0, IBLK), :], idxh_blk)
        pltpu.sync_copy(poff_hbm.at[pl.ds(row0, IBLK), :], poff_blk)
        fire(0, rows_a, sem_a)

        def pair_body(i, carry2):
            c0 = 2 * i
            c1 = 2 * i + 1
            fire(c1, rows_b, sem_b)
            wait_gather(c0, rows_a, sem_a)

            @pl.when(i > 0)
            def _():
                out_copy(row0, c0, out_a, sem_oa).wait()
            accum(c0, rows_a, out_a)
            out_copy(row0, c0, out_a, sem_oa).start()

            @pl.when(i < CPB // 2 - 1)
            def _():
                fire(c0 + 2, rows_a, sem_a)

            wait_gather(c1, rows_b, sem_b)

            @pl.when(i > 0)
            def _():
                out_copy(row0, c1, out_b, sem_ob).wait()
            accum(c1, rows_b, out_b)
            out_copy(row0, c1, out_b, sem_ob).start()
            return carry2

        lax.fori_loop(0, CPB // 2, pair_body, 0)
        out_copy(row0, 0, out_a, sem_oa).wait()
        out_copy(row0, 0, out_b, sem_ob).wait()
        return carry

    lax.fori_loop(0, N_BLK, block_body, 0)


BN = 4096


def _tr_body(t1_ref, t2_ref, o_ref):
    o_ref[...] = jnp.concatenate([t1_ref[...].T, t2_ref[...].T], axis=1)


def _repack(tbl_t):
    return pl.pallas_call(
        _tr_body,
        grid=(HALF // BN,),
        in_specs=[
            pl.BlockSpec((EMB, BN), lambda i: (0, i)),
            pl.BlockSpec(
                (EMB, BN),
                lambda i: (0, jnp.minimum(i + HALF // BN, 1000000 // BN)),
            ),
        ],
        out_specs=pl.BlockSpec((BN, 2 * EMB), lambda i: (i, 0)),
        out_shape=jax.ShapeDtypeStruct((HALF, 2 * EMB), jnp.float32),
    )(tbl_t, tbl_t)


BM = 1024


def _mm_body(m_ref, w_ref, b_ref, o_ref):
    o_ref[...] = jnp.tanh(
        jnp.dot(m_ref[...], w_ref[...], preferred_element_type=jnp.float32)
        + b_ref[...]
    )


def _matmul(m, w, b2d):
    return pl.pallas_call(
        _mm_body,
        grid=(B // BM,),
        in_specs=[
            pl.BlockSpec((BM, EMB), lambda i: (i, 0)),
            pl.BlockSpec((EMB, OUT), lambda i: (0, 0)),
            pl.BlockSpec((1, OUT), lambda i: (0, 0)),
        ],
        out_specs=pl.BlockSpec((BM, OUT), lambda i: (i, 0)),
        out_shape=jax.ShapeDtypeStruct((B, OUT), jnp.float32),
    )(m, w, b2d)


def kernel(input_a, emb_table, W, b):
    tbl2 = _repack(emb_table.T)
    hi = (input_a >= HALF).astype(jnp.int32)
    idx_half = input_a - hi * HALF
    poff = jnp.pad(hi << 6, ((0, 0), (0, LP - L)))
    m = _pool(idx_half, poff, tbl2)
    return _matmul(m, W, b.reshape(1, OUT))


# speedup vs baseline: 2.2375x; 1.3120x over previous
import functools

import jax
import jax.numpy as jnp
from jax import lax
from jax.experimental import pallas as pl
from jax.experimental.pallas import tpu as pltpu
from jax.experimental.pallas import tpu_sc as plsc

B = 16384
L = 50
EMB = 64
OUT = 1024
HALF = 512000
LP = 64

NC = 2
NS = 16
NW = NC * NS
B_PER_W = B // NW
CHUNK = 8
IBLK = 128
N_BLK = B_PER_W // IBLK
CPB = IBLK // CHUNK
VECS = EMB // 16

_mesh = plsc.VectorSubcoreMesh(core_axis_name="c", subcore_axis_name="s")


@functools.partial(
    pl.kernel,
    mesh=_mesh,
    out_type=jax.ShapeDtypeStruct((B, EMB), jnp.float32),
    scratch_types=[
        pltpu.VMEM((IBLK, L), jnp.int32),
        pltpu.VMEM((CHUNK, L, EMB), jnp.float32),
        pltpu.VMEM((CHUNK, L, EMB), jnp.float32),
        pltpu.VMEM((CHUNK, EMB), jnp.float32),
        pltpu.VMEM((CHUNK, EMB), jnp.float32),
        pltpu.SemaphoreType.DMA,
        pltpu.SemaphoreType.DMA,
        pltpu.SemaphoreType.DMA,
        pltpu.SemaphoreType.DMA,
    ],
    compiler_params=pltpu.CompilerParams(use_tc_tiling_on_sc=False),
)
def _pool(idx_hbm, tbl_hbm, out_hbm,
          idx_blk, rows_a, rows_b, out_a, out_b,
          sem_a, sem_b, sem_oa, sem_ob):
    wid = lax.axis_index("s") * NC + lax.axis_index("c")
    base = wid * B_PER_W

    def fire(c, rows_v, sem):
        for j in range(CHUNK):
            pltpu.async_copy(
                tbl_hbm.at[idx_blk.at[c * CHUNK + j]], rows_v.at[j], sem)

    def wait_gather(c, rows_v, sem):
        for j in range(CHUNK):
            pltpu.make_async_copy(
                tbl_hbm.at[idx_blk.at[c * CHUNK + j]], rows_v.at[j],
                sem).wait()

    def out_copy(row0, c, out_v, sem):
        return pltpu.make_async_copy(
            out_v, out_hbm.at[pl.ds(row0 + c * CHUNK, CHUNK), :], sem)

    def accum(c, rows_v, out_v):
        for j in range(CHUNK):
            def lsum(l, accs):
                return tuple(
                    accs[k] + rows_v[j, l, pl.ds(k * 16, 16)]
                    for k in range(VECS)
                )
            acc = lax.fori_loop(
                0, L, lsum,
                tuple(jnp.zeros((16,), jnp.float32) for _ in range(VECS)),
            )
            for k in range(VECS):
                out_v[j, pl.ds(k * 16, 16)] = acc[k] * (1.0 / L)

    def block_body(blk, carry):
        row0 = base + blk * IBLK
        pltpu.sync_copy(idx_hbm.at[pl.ds(row0, IBLK), :], idx_blk)
        fire(0, rows_a, sem_a)

        def pair_body(i, carry2):
            c0 = 2 * i
            c1 = 2 * i + 1
            fire(c1, rows_b, sem_b)
            wait_gather(c0, rows_a, sem_a)

            @pl.when(i > 0)
            def _():
                out_copy(row0, c0, out_a, sem_oa).wait()
            accum(c0, rows_a, out_a)
            out_copy(row0, c0, out_a, sem_oa).start()

            @pl.when(i < CPB // 2 - 1)
            def _():
                fire(c0 + 2, rows_a, sem_a)

            wait_gather(c1, rows_b, sem_b)

            @pl.when(i > 0)
            def _():
                out_copy(row0, c1, out_b, sem_ob).wait()
            accum(c1, rows_b, out_b)
            out_copy(row0, c1, out_b, sem_ob).start()
            return carry2

        lax.fori_loop(0, CPB // 2, pair_body, 0)
        out_copy(row0, 0, out_a, sem_oa).wait()
        out_copy(row0, 0, out_b, sem_ob).wait()
        return carry

    lax.fori_loop(0, N_BLK, block_body, 0)


BN = 4096


def _tr_body(t1_ref, t2_ref, o_ref):
    o_ref[...] = jnp.concatenate([t1_ref[...].T, t2_ref[...].T], axis=1)


def _repack(tbl_t):
    return pl.pallas_call(
        _tr_body,
        grid=(HALF // BN,),
        in_specs=[
            pl.BlockSpec((EMB, BN), lambda i: (0, i)),
            pl.BlockSpec(
                (EMB, BN),
                lambda i: (0, jnp.minimum(i + HALF // BN, 1000000 // BN)),
            ),
        ],
        out_specs=pl.BlockSpec((BN, 2 * EMB), lambda i: (i, 0)),
        out_shape=jax.ShapeDtypeStruct((HALF, 2 * EMB), jnp.float32),
    )(tbl_t, tbl_t)


BM = 1024


def _mm_body(m_ref, w_ref, b_ref, o_ref):
    o_ref[...] = jnp.tanh(
        jnp.dot(m_ref[...], w_ref[...], preferred_element_type=jnp.float32)
        + b_ref[...]
    )


def _matmul(m, w, b2d):
    return pl.pallas_call(
        _mm_body,
        grid=(B // BM,),
        in_specs=[
            pl.BlockSpec((BM, EMB), lambda i: (i, 0)),
            pl.BlockSpec((EMB, OUT), lambda i: (0, 0)),
            pl.BlockSpec((1, OUT), lambda i: (0, 0)),
        ],
        out_specs=pl.BlockSpec((BM, OUT), lambda i: (i, 0)),
        out_shape=jax.ShapeDtypeStruct((B, OUT), jnp.float32),
    )(m, w, b2d)


def kernel(input_a, emb_table, W, b):
    tbl2 = _repack(emb_table.T)
    tbl_lin = tbl2.reshape(2 * HALF, EMB)
    hi = (input_a >= HALF).astype(jnp.int32)
    idx_lin = 2 * (input_a - hi * HALF) + hi
    m = _pool(idx_lin, tbl_lin)
    return _matmul(m, W, b.reshape(1, OUT))
